# Initial kernel scaffold; baseline (speedup 1.0000x reference)
import jax
import jax.numpy as jnp
from jax.experimental import pallas as pl


def kernel(x, edge_index, W1, b1, W2, b2):
    raise NotImplementedError("write your pallas kernel here")


# speedup vs baseline: 147.1411x; 147.1411x over previous
import functools

import jax
import jax.numpy as jnp
from jax import lax
from jax.experimental import pallas as pl
from jax.experimental.pallas import tpu as pltpu
from jax.experimental.pallas import tpu_sc as plsc

N_NODES = 100000
HALF = N_NODES // 2
D_ROWS = 50176
D_TZ = D_ROWS // 16
S_ROWS = 50048
S_TZ = S_ROWS // 16
S_ZC = 184
C = 32
CHUNK = 128
NC, NS = 2, 16

_mesh = plsc.VectorSubcoreMesh(core_axis_name="c", subcore_axis_name="s")


I32 = jnp.int32


def _fori(n, body):
    lax.fori_loop(I32(0), I32(n), body, I32(0))


def _zero_rows(zbuf):
    z16 = jnp.zeros((16,), jnp.float32)

    def body(i, _):
        zbuf[i, pl.ds(I32(0), 16)] = z16
        zbuf[i, pl.ds(I32(16), 16)] = z16
        return _

    _fori(zbuf.shape[0], body)


def _local_dst(dst_raw, dst_loc, base, limit):
    lim = I32(limit)

    def body(k, _):
        d = dst_raw[pl.ds(k * I32(16), 16)]
        loc = d - base
        ok = (loc >= I32(0)) & (loc < lim)
        dst_loc[I32(0), pl.ds(k * I32(16), 16)] = jnp.where(ok, loc, lim)
        return _

    _fori(CHUNK // 16, body)


def _sc_deg(dst_pad):
    e_pad = dst_pad.shape[0]
    ept = e_pad // NS
    nch = ept // CHUNK

    @functools.partial(
        pl.kernel,
        out_type=jax.ShapeDtypeStruct((NC * D_ROWS,), jnp.float32),
        mesh=_mesh,
        scratch_types=[
            pltpu.VMEM((CHUNK,), jnp.int32),
            pltpu.VMEM((1, CHUNK), jnp.int32),
            pltpu.VMEM((CHUNK,), jnp.float32),
            pltpu.VMEM((D_TZ,), jnp.float32),
            pltpu.VMEM_SHARED((D_ROWS,), jnp.float32),
        ],
    )
    def k(dst_hbm, deg_hbm, dst_raw, dst_loc, ones_v, zbuf, acc):
        c = lax.axis_index("c").astype(jnp.int32)
        s = lax.axis_index("s").astype(jnp.int32)
        base = c * I32(HALF)

        def zb(i, _):
            zbuf[pl.ds(i * I32(16), 16)] = jnp.zeros((16,), jnp.float32)
            return _

        _fori(D_TZ // 16, zb)

        def ob(i, _):
            ones_v[pl.ds(i * I32(16), 16)] = jnp.full((16,), 1.0, jnp.float32)
            return _

        _fori(CHUNK // 16, ob)

        pltpu.sync_copy(zbuf, acc.at[pl.ds(s * I32(D_TZ), D_TZ)])
        plsc.subcore_barrier()

        def chunk(j, _):
            off = s * I32(ept) + j * I32(CHUNK)
            pltpu.sync_copy(dst_hbm.at[pl.ds(off, CHUNK)], dst_raw)
            _local_dst(dst_raw, dst_loc, base, HALF)
            pltpu.sync_copy(ones_v, acc.at[dst_loc.at[I32(0)]], add=True)
            return _

        _fori(nch, chunk)
        plsc.subcore_barrier()

        pltpu.sync_copy(acc.at[pl.ds(s * I32(D_TZ), D_TZ)], zbuf)
        pltpu.sync_copy(zbuf,
                        deg_hbm.at[pl.ds(c * I32(D_ROWS) + s * I32(D_TZ),
                                         D_TZ)])

    return k(dst_pad)


def _sc_spmm(feat, src_pad, dst_pad):
    e_pad = src_pad.shape[0]
    ept = e_pad // NS
    nch = ept // CHUNK

    @functools.partial(
        pl.kernel,
        out_type=jax.ShapeDtypeStruct((NC * S_ROWS, C), jnp.float32),
        mesh=_mesh,
        compiler_params=pltpu.CompilerParams(use_tc_tiling_on_sc=False),
        scratch_types=[
            pltpu.VMEM((CHUNK,), jnp.int32),
            pltpu.VMEM((CHUNK,), jnp.int32),
            pltpu.VMEM((1, CHUNK), jnp.int32),
            pltpu.VMEM((CHUNK, C), jnp.float32),
            pltpu.VMEM((S_ZC, C), jnp.float32),
            pltpu.VMEM_SHARED((S_ROWS, C), jnp.float32),
            pltpu.SemaphoreType.DMA,
        ],
    )
    def k(feat_hbm, src_hbm, dst_hbm, out_hbm,
          src_v, dst_raw, dst_loc, rows_v, zbuf, acc, sem):
        c = lax.axis_index("c").astype(jnp.int32)
        s = lax.axis_index("s").astype(jnp.int32)
        base = c * I32(HALF)

        _zero_rows(zbuf)

        def zc(j, _):
            pltpu.sync_copy(zbuf,
                            acc.at[pl.ds(s * I32(S_TZ) + j * I32(S_ZC),
                                         S_ZC), :])
            return _

        _fori(S_TZ // S_ZC, zc)
        plsc.subcore_barrier()

        def chunk(j, _):
            off = s * I32(ept) + j * I32(CHUNK)
            pltpu.sync_copy(src_hbm.at[pl.ds(off, CHUNK)], src_v)
            pltpu.sync_copy(dst_hbm.at[pl.ds(off, CHUNK)], dst_raw)
            g = pltpu.async_copy(feat_hbm.at[src_v], rows_v, sem)
            _local_dst(dst_raw, dst_loc, base, HALF)
            g.wait()
            pltpu.sync_copy(rows_v, acc.at[dst_loc.at[I32(0)]], add=True)
            return _

        _fori(nch, chunk)
        plsc.subcore_barrier()

        def wb(j, _):
            r0 = s * I32(S_TZ) + j * I32(S_ZC)
            pltpu.sync_copy(acc.at[pl.ds(r0, S_ZC), :], zbuf)
            pltpu.sync_copy(zbuf,
                            out_hbm.at[pl.ds(c * I32(S_ROWS) + r0, S_ZC), :])
            return _

        _fori(S_TZ // S_ZC, wb)

    return k(feat, src_pad, dst_pad)


def _tc_prescale(deg2, xp):
    B = 2000
    grid = N_NODES // B

    def body(d_ref, x_ref, dinv_ref, y_ref):
        dinv = lax.rsqrt(d_ref[...] + 1.0)
        dinv_ref[...] = dinv
        y_ref[...] = x_ref[...] * dinv

    return pl.pallas_call(
        body,
        grid=(grid,),
        in_specs=[
            pl.BlockSpec((B, 1), lambda i: (i, I32(0))),
            pl.BlockSpec((B, C), lambda i: (i, I32(0))),
        ],
        out_specs=[
            pl.BlockSpec((B, 1), lambda i: (i, I32(0))),
            pl.BlockSpec((B, C), lambda i: (i, I32(0))),
        ],
        out_shape=[
            jax.ShapeDtypeStruct((N_NODES, 1), jnp.float32),
            jax.ShapeDtypeStruct((N_NODES, C), jnp.float32),
        ],
    )(deg2, xp)


def _tc_layer(agg, y, dinv2, W, b2):
    B = 2000
    grid = N_NODES // B

    def body(a_ref, y_ref, d_ref, w_ref, b_ref, h_ref, yn_ref):
        dinv = d_ref[...]
        sm = (a_ref[...] + y_ref[...]) * dinv
        h = jnp.dot(sm, w_ref[...], preferred_element_type=jnp.float32)
        h = jnp.maximum(h + b_ref[...], 0.0)
        h_ref[...] = h
        yn_ref[...] = h * dinv

    return pl.pallas_call(
        body,
        grid=(grid,),
        in_specs=[
            pl.BlockSpec((B, C), lambda i: (i, I32(0))),
            pl.BlockSpec((B, C), lambda i: (i, I32(0))),
            pl.BlockSpec((B, 1), lambda i: (i, I32(0))),
            pl.BlockSpec((C, C), lambda i: (I32(0), I32(0))),
            pl.BlockSpec((1, C), lambda i: (I32(0), I32(0))),
        ],
        out_specs=[
            pl.BlockSpec((B, C), lambda i: (i, I32(0))),
            pl.BlockSpec((B, C), lambda i: (i, I32(0))),
        ],
        out_shape=[
            jax.ShapeDtypeStruct((N_NODES, C), jnp.float32),
            jax.ShapeDtypeStruct((N_NODES, C), jnp.float32),
        ],
    )(agg, y, dinv2, W, b2)


def kernel(x, edge_index, W1, b1, W2, b2):
    n, e = x.shape[0], edge_index.shape[1]
    ei = edge_index.astype(jnp.int32)
    ept = ((e + NS * CHUNK - 1) // (NS * CHUNK)) * CHUNK
    pad = ept * NS - e
    src_p = jnp.concatenate([ei[0], jnp.zeros((pad,), jnp.int32)])
    dst_p = jnp.concatenate([ei[1], jnp.full((pad,), -1, jnp.int32)])

    xp = jnp.pad(x.astype(jnp.float32), ((0, 0), (0, C - x.shape[1])))
    W1p = jnp.pad(W1.astype(jnp.float32), ((0, C - W1.shape[0]), (0, 0)))
    W2f = W2.astype(jnp.float32)
    b1r = b1.astype(jnp.float32).reshape(1, C)
    b2r = b2.astype(jnp.float32).reshape(1, C)

    degp = _sc_deg(dst_p)
    deg = jnp.concatenate([degp[:HALF], degp[D_ROWS:D_ROWS + HALF]])
    dinv2, y1 = _tc_prescale(deg.reshape(n, 1), xp)
    agg1p = _sc_spmm(y1, src_p, dst_p)
    agg1 = jnp.concatenate([agg1p[:HALF], agg1p[S_ROWS:S_ROWS + HALF]])
    _, y2 = _tc_layer(agg1, y1, dinv2, W1p, b1r)
    agg2p = _sc_spmm(y2, src_p, dst_p)
    agg2 = jnp.concatenate([agg2p[:HALF], agg2p[S_ROWS:S_ROWS + HALF]])
    h2, _ = _tc_layer(agg2, y2, dinv2, W2f, b2r)
    return h2.astype(jnp.float64)


# speedup vs baseline: 279.6079x; 1.9003x over previous
import functools

import jax
import jax.numpy as jnp
from jax import lax
from jax.experimental import pallas as pl
from jax.experimental.pallas import tpu as pltpu
from jax.experimental.pallas import tpu_sc as plsc

N_NODES = 100000
ACC_ROWS = 100096
TZ = ACC_ROWS // 16
ZC = 184
DZC = 3128
CH = 16
C = 32
SUP = 512
NSUB = SUP // 128
NC, NS = 2, 16
E_PAD = 1605632

_mesh = plsc.VectorSubcoreMesh(core_axis_name="c", subcore_axis_name="s")

I32 = jnp.int32


def _fori(n, body):
    lax.fori_loop(I32(0), I32(n), body, I32(0))


def _sc_deg(dst2d):
    rows_pt = E_PAD // 128 // (NC * NS)
    g_total = rows_pt // NSUB

    @functools.partial(
        pl.kernel,
        out_type=jax.ShapeDtypeStruct((NC * ACC_ROWS,), jnp.float32),
        mesh=_mesh,
        scratch_types=[
            pltpu.VMEM((2, NSUB, 128), jnp.int32),
            pltpu.VMEM((128,), jnp.float32),
            pltpu.VMEM((DZC,), jnp.float32),
            pltpu.VMEM_SHARED((ACC_ROWS,), jnp.float32),
            pltpu.SemaphoreType.DMA,
            pltpu.SemaphoreType.DMA,
        ],
    )
    def k(dst_hbm, deg_hbm, dstb, ones_v, zbuf, acc, sem0, sem1):
        c = lax.axis_index("c").astype(jnp.int32)
        s = lax.axis_index("s").astype(jnp.int32)
        rb0 = (c * I32(NS) + s) * I32(rows_pt)

        def zb(i, carry):
            zbuf[pl.ds(i * I32(16), 16)] = jnp.zeros((16,), jnp.float32)
            return carry

        _fori(DZC // 16, zb)

        def ob(i, carry):
            ones_v[pl.ds(i * I32(16), 16)] = jnp.full((16,), 1.0, jnp.float32)
            return carry

        _fori(128 // 16, ob)

        pltpu.sync_copy(zbuf, acc.at[pl.ds(s * I32(TZ), DZC)])
        pltpu.sync_copy(zbuf, acc.at[pl.ds(s * I32(TZ) + I32(DZC), DZC)])
        plsc.subcore_barrier()

        sems = [sem0, sem1]

        def wait_sc(b):
            for j in range(NSUB):
                pltpu.make_async_copy(ones_v,
                                      acc.at[dstb.at[I32(b), I32(j)]],
                                      sems[b]).wait()

        def step(g, b):
            nb = 1 - b
            for j in range(NSUB):
                pltpu.async_copy(ones_v, acc.at[dstb.at[I32(b), I32(j)]],
                                 sems[b], add=True)

            @pl.when(g < I32(g_total - 1))
            def _():
                @pl.when(g >= I32(1))
                def _():
                    wait_sc(nb)

                pltpu.sync_copy(
                    dst_hbm.at[pl.ds(rb0 + (g + I32(1)) * I32(NSUB),
                                     NSUB), :],
                    dstb.at[I32(nb)])

        pltpu.sync_copy(dst_hbm.at[pl.ds(rb0, NSUB), :], dstb.at[I32(0)])

        def chunk(g, carry):
            b = lax.rem(g, I32(2))

            @pl.when(b == I32(0))
            def _():
                step(g, 0)

            @pl.when(b == I32(1))
            def _():
                step(g, 1)

            return carry

        _fori(g_total, chunk)
        if g_total > 1:
            wait_sc((g_total - 2) % 2)
        wait_sc((g_total - 1) % 2)
        plsc.subcore_barrier()

        def wb(j, carry):
            off = s * I32(TZ) + j * I32(DZC)
            pltpu.sync_copy(acc.at[pl.ds(off, DZC)], zbuf)
            pltpu.sync_copy(zbuf,
                            deg_hbm.at[pl.ds(c * I32(ACC_ROWS) + off, DZC)])
            return carry

        _fori(TZ // DZC, wb)

    return k(dst2d)


def _sc_spmm(f0, f1, src2d, dst2d):
    rows_pt = E_PAD // 128 // NS
    g_total = rows_pt // NSUB

    @functools.partial(
        pl.kernel,
        out_type=jax.ShapeDtypeStruct((NC, ACC_ROWS, CH), jnp.float32),
        mesh=_mesh,
        compiler_params=pltpu.CompilerParams(use_tc_tiling_on_sc=False),
        scratch_types=[
            pltpu.VMEM((2, NSUB, 128), jnp.int32),
            pltpu.VMEM((2, NSUB, 128), jnp.int32),
            pltpu.VMEM((2, SUP, CH), jnp.float32),
            pltpu.VMEM((ZC, CH), jnp.float32),
            pltpu.VMEM_SHARED((ACC_ROWS, CH), jnp.float32),
            pltpu.SemaphoreType.DMA,
            pltpu.SemaphoreType.DMA,
            pltpu.SemaphoreType.DMA,
            pltpu.SemaphoreType.DMA,
        ],
    )
    def k(f0_hbm, f1_hbm, src_hbm, dst_hbm, out_hbm,
          srcb, dstb, rows, zbuf, acc, gsem0, gsem1, ssem0, ssem1):
        c = lax.axis_index("c").astype(jnp.int32)
        s = lax.axis_index("s").astype(jnp.int32)
        rb0 = s * I32(rows_pt)
        gsems = [gsem0, gsem1]
        ssems = [ssem0, ssem1]

        def zrow(i, carry):
            zbuf[i, pl.ds(I32(0), 16)] = jnp.zeros((16,), jnp.float32)
            return carry

        _fori(ZC, zrow)

        def zc(j, carry):
            pltpu.sync_copy(zbuf, acc.at[pl.ds(s * I32(TZ) + j * I32(ZC),
                                               ZC), :])
            return carry

        _fori(TZ // ZC, zc)
        plsc.subcore_barrier()

        def wait_g(b):
            @pl.when(c == I32(0))
            def _():
                for j in range(NSUB):
                    pltpu.make_async_copy(
                        f0_hbm.at[srcb.at[I32(b), I32(j)]],
                        rows.at[I32(b), pl.ds(I32(j * 128), 128), :],
                        gsems[b]).wait()

            @pl.when(c == I32(1))
            def _():
                for j in range(NSUB):
                    pltpu.make_async_copy(
                        f1_hbm.at[srcb.at[I32(b), I32(j)]],
                        rows.at[I32(b), pl.ds(I32(j * 128), 128), :],
                        gsems[b]).wait()

        def wait_s(b):
            for j in range(NSUB):
                pltpu.make_async_copy(
                    rows.at[I32(b), pl.ds(I32(j * 128), 128), :],
                    acc.at[dstb.at[I32(b), I32(j)]],
                    ssems[b]).wait()

        def load_idx(g, b):
            pltpu.sync_copy(src_hbm.at[pl.ds(rb0 + g * I32(NSUB), NSUB), :],
                            srcb.at[I32(b)])
            pltpu.sync_copy(dst_hbm.at[pl.ds(rb0 + g * I32(NSUB), NSUB), :],
                            dstb.at[I32(b)])

        def fire_g(b):
            @pl.when(c == I32(0))
            def _():
                for j in range(NSUB):
                    pltpu.async_copy(
                        f0_hbm.at[srcb.at[I32(b), I32(j)]],
                        rows.at[I32(b), pl.ds(I32(j * 128), 128), :],
                        gsems[b])

            @pl.when(c == I32(1))
            def _():
                for j in range(NSUB):
                    pltpu.async_copy(
                        f1_hbm.at[srcb.at[I32(b), I32(j)]],
                        rows.at[I32(b), pl.ds(I32(j * 128), 128), :],
                        gsems[b])

        def fire_s(b):
            for j in range(NSUB):
                pltpu.async_copy(
                    rows.at[I32(b), pl.ds(I32(j * 128), 128), :],
                    acc.at[dstb.at[I32(b), I32(j)]],
                    ssems[b], add=True)

        def step(g, b):
            nb = 1 - b
            wait_g(b)
            fire_s(b)

            @pl.when(g < I32(g_total - 1))
            def _():
                @pl.when(g >= I32(1))
                def _():
                    wait_s(nb)

                load_idx(g + I32(1), nb)
                fire_g(nb)

        load_idx(I32(0), 0)
        fire_g(0)

        def chunk(g, carry):
            b = lax.rem(g, I32(2))

            @pl.when(b == I32(0))
            def _():
                step(g, 0)

            @pl.when(b == I32(1))
            def _():
                step(g, 1)

            return carry

        _fori(g_total, chunk)
        if g_total > 1:
            wait_s((g_total - 2) % 2)
        wait_s((g_total - 1) % 2)
        plsc.subcore_barrier()

        def wb(j, carry):
            off = s * I32(TZ) + j * I32(ZC)
            pltpu.sync_copy(acc.at[pl.ds(off, ZC), :], zbuf)
            pltpu.sync_copy(zbuf, out_hbm.at[c, pl.ds(off, ZC), :])
            return carry

        _fori(TZ // ZC, wb)

    return k(f0, f1, src2d, dst2d)


def _tc_prescale(d0, d1, xp):
    B = 2000
    grid = N_NODES // B

    def body(d0_ref, d1_ref, x_ref, dinv_ref, y0_ref, y1_ref):
        dinv = lax.rsqrt(d0_ref[...] + d1_ref[...] + 1.0)
        dinv_ref[...] = dinv
        y = x_ref[...] * dinv
        y0_ref[...] = y[:, :CH]
        y1_ref[...] = y[:, CH:]

    return pl.pallas_call(
        body,
        grid=(grid,),
        in_specs=[
            pl.BlockSpec((B, 1), lambda i: (i, I32(0))),
            pl.BlockSpec((B, 1), lambda i: (i, I32(0))),
            pl.BlockSpec((B, C), lambda i: (i, I32(0))),
        ],
        out_specs=[
            pl.BlockSpec((B, 1), lambda i: (i, I32(0))),
            pl.BlockSpec((B, CH), lambda i: (i, I32(0))),
            pl.BlockSpec((B, CH), lambda i: (i, I32(0))),
        ],
        out_shape=[
            jax.ShapeDtypeStruct((N_NODES, 1), jnp.float32),
            jax.ShapeDtypeStruct((N_NODES, CH), jnp.float32),
            jax.ShapeDtypeStruct((N_NODES, CH), jnp.float32),
        ],
    )(d0, d1, xp)


def _tc_layer(a0, a1, y0, y1, dinv2, W, b2):
    B = 2000
    grid = N_NODES // B

    def body(a0_ref, a1_ref, y0_ref, y1_ref, d_ref, w_ref, b_ref,
             h_ref, z0_ref, z1_ref):
        dinv = d_ref[...]
        sm = jnp.concatenate(
            [a0_ref[...] + y0_ref[...], a1_ref[...] + y1_ref[...]],
            axis=1) * dinv
        h = jnp.dot(sm, w_ref[...], preferred_element_type=jnp.float32)
        h = jnp.maximum(h + b_ref[...], 0.0)
        h_ref[...] = h
        z0_ref[...] = h[:, :CH] * dinv
        z1_ref[...] = h[:, CH:] * dinv

    return pl.pallas_call(
        body,
        grid=(grid,),
        in_specs=[
            pl.BlockSpec((B, CH), lambda i: (i, I32(0))),
            pl.BlockSpec((B, CH), lambda i: (i, I32(0))),
            pl.BlockSpec((B, CH), lambda i: (i, I32(0))),
            pl.BlockSpec((B, CH), lambda i: (i, I32(0))),
            pl.BlockSpec((B, 1), lambda i: (i, I32(0))),
            pl.BlockSpec((C, C), lambda i: (I32(0), I32(0))),
            pl.BlockSpec((1, C), lambda i: (I32(0), I32(0))),
        ],
        out_specs=[
            pl.BlockSpec((B, C), lambda i: (i, I32(0))),
            pl.BlockSpec((B, CH), lambda i: (i, I32(0))),
            pl.BlockSpec((B, CH), lambda i: (i, I32(0))),
        ],
        out_shape=[
            jax.ShapeDtypeStruct((N_NODES, C), jnp.float32),
            jax.ShapeDtypeStruct((N_NODES, CH), jnp.float32),
            jax.ShapeDtypeStruct((N_NODES, CH), jnp.float32),
        ],
    )(a0, a1, y0, y1, dinv2, W, b2)


def kernel(x, edge_index, W1, b1, W2, b2):
    n, e = x.shape[0], edge_index.shape[1]
    ei = edge_index.astype(jnp.int32)
    pad = E_PAD - e
    src_p = jnp.concatenate([ei[0], jnp.zeros((pad,), jnp.int32)])
    dst_p = jnp.concatenate([ei[1], jnp.full((pad,), n, jnp.int32)])
    src2d = src_p.reshape(E_PAD // 128, 128)
    dst2d = dst_p.reshape(E_PAD // 128, 128)

    xp = jnp.pad(x.astype(jnp.float32), ((0, 0), (0, C - x.shape[1])))
    W1p = jnp.pad(W1.astype(jnp.float32), ((0, C - W1.shape[0]), (0, 0)))
    W2f = W2.astype(jnp.float32)
    b1r = b1.astype(jnp.float32).reshape(1, C)
    b2r = b2.astype(jnp.float32).reshape(1, C)

    degp = _sc_deg(dst2d)
    d0 = degp[:n].reshape(n, 1)
    d1 = degp[ACC_ROWS:ACC_ROWS + n].reshape(n, 1)
    dinv2, y0, y1 = _tc_prescale(d0, d1, xp)
    agg1 = _sc_spmm(y0, y1, src2d, dst2d)
    _, z0, z1 = _tc_layer(agg1[0], agg1[1], y0, y1, dinv2, W1p, b1r)
    agg2 = _sc_spmm(z0, z1, src2d, dst2d)
    h2, _, _ = _tc_layer(agg2[0], agg2[1], z0, z1, dinv2, W2f, b2r)
    return h2.astype(jnp.float64)
